# baseline (speedup 1.0000x reference)
import functools

import jax
import jax.numpy as jnp
from jax import lax
from jax.experimental import pallas as pl
from jax.experimental.pallas import tpu as pltpu
from jax.experimental.pallas import tpu_sc as plsc

_B, _CTX, _D, _V = 4096, 20, 64, 100000
_NC, _NS = 2, 16
_NW = _NC * _NS
_BPW = _B // _NW
_CHUNK = 4
_NCHUNK = _BPW // _CHUNK
_IDXC = _CHUNK * _CTX

_mesh = plsc.VectorSubcoreMesh(core_axis_name="c", subcore_axis_name="s")


@functools.partial(
    pl.kernel,
    mesh=_mesh,
    out_type=jax.ShapeDtypeStruct((_B, _D), jnp.float32),
    scratch_types=[
        pltpu.VMEM((_BPW * _CTX,), jnp.int32),
        pltpu.VMEM((_IDXC, _D), jnp.float32),
        pltpu.VMEM((_BPW, _D), jnp.float32),
        pltpu.SemaphoreType.DMA,
    ],
    compiler_params=pltpu.CompilerParams(use_tc_tiling_on_sc=False),
)
def _pool(idx_hbm, table_hbm, out_hbm, idx_v, rows_v, pooled_v, sem):
    wid = lax.axis_index("s") * _NC + lax.axis_index("c")
    base = wid * _BPW
    idx_off = pl.multiple_of(base * _CTX, 8)
    pltpu.sync_copy(idx_hbm.at[pl.ds(idx_off, _BPW * _CTX)], idx_v)

    def chunk(j, carry):
        off = pl.multiple_of(j * _IDXC, 8)
        pltpu.async_copy(
            table_hbm.at[idx_v.at[pl.ds(off, _IDXC)]], rows_v, sem
        ).wait()
        for t in range(_CHUNK):
            for k in range(_D // 16):
                acc = rows_v[t * _CTX, pl.ds(k * 16, 16)]
                for cc in range(1, _CTX):
                    acc = acc + rows_v[t * _CTX + cc, pl.ds(k * 16, 16)]
                pooled_v[j * _CHUNK + t, pl.ds(k * 16, 16)] = acc * (1.0 / _CTX)
        return carry

    lax.fori_loop(0, _NCHUNK, chunk, 0)
    pltpu.sync_copy(pooled_v, out_hbm.at[pl.ds(pl.multiple_of(base, 8), _BPW)])


_BB = 1024
_VB = 2048


def _proj_body(p_ref, w_ref, b_ref, o_ref):
    o_ref[...] = lax.dot_general(
        p_ref[...], w_ref[...], (((1,), (1,)), ((), ())),
        preferred_element_type=jnp.float32,
    ) + b_ref[...]


@jax.jit
def _project(pooled, linear_w, bias2d):
    grid = (pl.cdiv(_V, _VB), _B // _BB)
    return pl.pallas_call(
        _proj_body,
        grid=grid,
        in_specs=[
            pl.BlockSpec((_BB, _D), lambda v, b: (b, 0)),
            pl.BlockSpec((_VB, _D), lambda v, b: (v, 0)),
            pl.BlockSpec((1, _VB), lambda v, b: (0, v)),
        ],
        out_specs=pl.BlockSpec((_BB, _VB), lambda v, b: (b, v)),
        out_shape=jax.ShapeDtypeStruct((_B, _V), jnp.float32),
    )(pooled, linear_w, bias2d)


def kernel(context_words_indices, embeddings, linear_w, linear_b):
    idx = jnp.asarray(context_words_indices, jnp.int32).reshape(-1)
    pooled = _pool(idx, embeddings)
    return _project(pooled, linear_w, linear_b.reshape(1, _V))


# speedup vs baseline: 1.0362x; 1.0362x over previous
import functools

import jax
import jax.numpy as jnp
from jax import lax
from jax.experimental import pallas as pl
from jax.experimental.pallas import tpu as pltpu
from jax.experimental.pallas import tpu_sc as plsc

_B, _CTX, _D, _V = 4096, 20, 64, 100000
_DP = 128
_NC, _NS = 2, 16
_NW = _NC * _NS
_BPW = _B // _NW
_CHUNK = 4
_NCHUNK = _BPW // _CHUNK
_IDXC = _CHUNK * _CTX

_mesh = plsc.VectorSubcoreMesh(core_axis_name="c", subcore_axis_name="s")


@functools.partial(
    pl.kernel,
    mesh=_mesh,
    out_type=jax.ShapeDtypeStruct((_B, _DP), jnp.float32),
    scratch_types=[
        pltpu.VMEM((_BPW * _CTX,), jnp.int32),
        pltpu.VMEM((2, _IDXC, _DP), jnp.float32),
        pltpu.VMEM((_BPW, _DP), jnp.float32),
        pltpu.SemaphoreType.DMA,
        pltpu.SemaphoreType.DMA,
    ],
)
def _pool(idx_hbm, table_hbm, out_hbm, idx_v, rows_v, pooled_v, sem0, sem1):
    wid = lax.axis_index("s") * _NC + lax.axis_index("c")
    base = wid * _BPW
    idx_off = pl.multiple_of(base * _CTX, 8)
    pltpu.sync_copy(idx_hbm.at[pl.ds(idx_off, _BPW * _CTX)], idx_v)
    sems = (sem0, sem1)
    zero = jnp.zeros((16,), jnp.float32)

    def fire(c, slot):
        off = pl.multiple_of(c * _IDXC, 8)
        pltpu.async_copy(
            table_hbm.at[idx_v.at[pl.ds(off, _IDXC)]], rows_v.at[slot],
            sems[slot])

    fire(0, 0)
    fire(1, 1)

    def chunk2(jj, carry):
        j = jj * 2
        for b in range(2):
            c = j + b
            pltpu.make_async_copy(
                table_hbm.at[idx_v.at[pl.ds(0, _IDXC)]], rows_v.at[b],
                sems[b]).wait()
            for t in range(_CHUNK):
                row = c * _CHUNK + t
                for k in range(_D // 16):
                    acc = rows_v[b, t * _CTX, pl.ds(k * 16, 16)]
                    for cc in range(1, _CTX):
                        acc = acc + rows_v[b, t * _CTX + cc, pl.ds(k * 16, 16)]
                    pooled_v[row, pl.ds(k * 16, 16)] = acc * (1.0 / _CTX)
                for k in range(_D // 16, _DP // 16):
                    pooled_v[row, pl.ds(k * 16, 16)] = zero

            @pl.when(c + 2 < _NCHUNK)
            def _():
                fire(c + 2, b)
        return carry

    lax.fori_loop(0, _NCHUNK // 2, chunk2, 0)
    pltpu.sync_copy(pooled_v, out_hbm.at[pl.ds(pl.multiple_of(base, 8), _BPW)])


_VB = 1024


def _proj_body(p_ref, w_ref, b_ref, o_ref):
    o_ref[...] = lax.dot_general(
        p_ref[...][:, :_D], w_ref[...], (((1,), (1,)), ((), ())),
        preferred_element_type=jnp.float32,
    ) + b_ref[...]


@jax.jit
def _project(pooled, linear_w, bias2d):
    return pl.pallas_call(
        _proj_body,
        grid=(pl.cdiv(_V, _VB),),
        in_specs=[
            pl.BlockSpec((_B, _DP), lambda v: (0, 0)),
            pl.BlockSpec((_VB, _D), lambda v: (v, 0)),
            pl.BlockSpec((1, _VB), lambda v: (0, v)),
        ],
        out_specs=pl.BlockSpec((_B, _VB), lambda v: (0, v)),
        out_shape=jax.ShapeDtypeStruct((_B, _V), jnp.float32),
    )(pooled, linear_w, bias2d)


def kernel(context_words_indices, embeddings, linear_w, linear_b):
    idx = jnp.asarray(context_words_indices, jnp.int32).reshape(-1)
    table128 = jnp.pad(embeddings, ((0, 0), (0, _DP - _D)))
    pooled128 = _pool(idx, table128)
    return _project(pooled128, linear_w, linear_b.reshape(1, _V))


# speedup vs baseline: 3.3738x; 3.2560x over previous
import functools

import jax
import jax.numpy as jnp
from jax import lax
from jax.experimental import pallas as pl
from jax.experimental.pallas import tpu as pltpu
from jax.experimental.pallas import tpu_sc as plsc

_B, _CTX, _D, _V = 4096, 20, 64, 100000
_DP = 128
_NC, _NS = 2, 16
_NW = _NC * _NS
_BPW = _B // _NW
_CHUNK = 4
_NCHUNK = _BPW // _CHUNK
_IDXC = _CHUNK * _CTX

_mesh = plsc.VectorSubcoreMesh(core_axis_name="c", subcore_axis_name="s")


_GRP = 16
_NGRP = _BPW // _GRP


@functools.partial(
    pl.kernel,
    mesh=_mesh,
    out_type=jax.ShapeDtypeStruct((_B, _DP), jnp.float32),
    scratch_types=[
        pltpu.VMEM((_CTX, _BPW), jnp.int32),
        pltpu.VMEM((2, _CTX * _GRP, _DP), jnp.float32),
        pltpu.VMEM((_BPW, _DP), jnp.float32),
        pltpu.SemaphoreType.DMA,
        pltpu.SemaphoreType.DMA,
    ],
)
def _pool(idxt_hbm, table_hbm, out_hbm, idx20_v, rows_v, pooled_v,
          sem0, sem1):
    wid = lax.axis_index("s") * _NC + lax.axis_index("c")
    base = wid * _BPW
    pltpu.sync_copy(idxt_hbm.at[:, pl.ds(pl.multiple_of(base, 8), _BPW)],
                    idx20_v)

    sems = (sem0, sem1)
    zero = jnp.zeros((16,), jnp.float32)

    def fire(g, slot):
        off = pl.multiple_of(g * _GRP, 8)
        for c in range(_CTX):
            pltpu.async_copy(
                table_hbm.at[idx20_v.at[c, pl.ds(off, _GRP)]],
                rows_v.at[slot, pl.ds(c * _GRP, _GRP)],
                sems[slot])

    def drain(slot):
        pltpu.make_async_copy(
            table_hbm.at[pl.ds(0, _CTX * _GRP), :],
            rows_v.at[slot],
            sems[slot]).wait()

    fire(0, 0)
    fire(1, 1)

    def grp2(jj, carry):
        j = jj * 2
        for b in range(2):
            g = j + b
            drain(b)
            for i in range(_GRP):
                row = g * _GRP + i
                for k in range(_D // 16):
                    acc = rows_v[b, i, pl.ds(k * 16, 16)]
                    for c in range(1, _CTX):
                        acc = acc + rows_v[b, c * _GRP + i, pl.ds(k * 16, 16)]
                    pooled_v[row, pl.ds(k * 16, 16)] = acc * (1.0 / _CTX)
                for k in range(_D // 16, _DP // 16):
                    pooled_v[row, pl.ds(k * 16, 16)] = zero

            @pl.when(g + 2 < _NGRP)
            def _():
                fire(g + 2, b)
        return carry

    lax.fori_loop(0, _NGRP // 2, grp2, 0)
    pltpu.sync_copy(pooled_v, out_hbm.at[pl.ds(pl.multiple_of(base, 8), _BPW)])


_VB = 1024


def _proj_body(p_ref, w_ref, b_ref, o_ref):
    ot = lax.dot_general(
        w_ref[...], p_ref[...][:, :_D], (((0,), (1,)), ((), ())),
        preferred_element_type=jnp.float32,
    )
    o_ref[...] = ot + b_ref[...].T


@jax.jit
def _project_t(pooled, w_t, bias2d):
    return pl.pallas_call(
        _proj_body,
        grid=(pl.cdiv(_V, _VB),),
        in_specs=[
            pl.BlockSpec((_B, _DP), lambda v: (0, 0)),
            pl.BlockSpec((_D, _VB), lambda v: (0, v)),
            pl.BlockSpec((1, _VB), lambda v: (0, v)),
        ],
        out_specs=pl.BlockSpec((_VB, _B), lambda v: (v, 0)),
        out_shape=jax.ShapeDtypeStruct((_V, _B), jnp.float32),
    )(pooled, w_t, bias2d)


def kernel(context_words_indices, embeddings, linear_w, linear_b):
    idxt = jnp.asarray(context_words_indices, jnp.int32).T
    table128 = jnp.pad(embeddings, ((0, 0), (0, _DP - _D)))
    pooled128 = _pool(idxt, table128)
    out_t = _project_t(pooled128, linear_w.T, linear_b.reshape(1, _V))
    return out_t.T


# speedup vs baseline: 3.4527x; 1.0234x over previous
import functools

import jax
import jax.numpy as jnp
from jax import lax
from jax.experimental import pallas as pl
from jax.experimental.pallas import tpu as pltpu
from jax.experimental.pallas import tpu_sc as plsc

_B, _CTX, _D, _V = 4096, 20, 64, 100000
_DP = 128
_NC, _NS = 2, 16
_NW = _NC * _NS
_BPW = _B // _NW
_GRP = 32
_NGRP = _BPW // _GRP

_mesh = plsc.VectorSubcoreMesh(core_axis_name="c", subcore_axis_name="s")


@functools.partial(
    pl.kernel,
    mesh=_mesh,
    out_type=jax.ShapeDtypeStruct((_B, _DP), jnp.float32),
    scratch_types=[
        pltpu.VMEM((_CTX, _BPW), jnp.int32),
        pltpu.VMEM((2, _CTX * _GRP, _D), jnp.float32),
        pltpu.VMEM((_BPW, _DP), jnp.float32),
        pltpu.SemaphoreType.DMA,
        pltpu.SemaphoreType.DMA,
    ],
    compiler_params=pltpu.CompilerParams(use_tc_tiling_on_sc=False),
)
def _pool(idx_hbm, table_hbm, out_hbm, idx20_v, rows_v, pooled_v,
          sem0, sem1):
    wid = lax.axis_index("s") * _NC + lax.axis_index("c")
    base = wid * _BPW
    for c in range(_CTX):
        pltpu.sync_copy(
            idx_hbm.at[pl.ds(pl.multiple_of(c * _B + base, 8), _BPW)],
            idx20_v.at[c])

    sems = (sem0, sem1)
    zero = jnp.zeros((16,), jnp.float32)

    def fire(g, slot):
        off = pl.multiple_of(g * _GRP, 8)
        for c in range(_CTX):
            pltpu.async_copy(
                table_hbm.at[idx20_v.at[c, pl.ds(off, _GRP)]],
                rows_v.at[slot, pl.ds(c * _GRP, _GRP)],
                sems[slot])

    def drain(slot):
        pltpu.make_async_copy(
            table_hbm.at[pl.ds(0, _CTX * _GRP), :],
            rows_v.at[slot],
            sems[slot]).wait()

    fire(0, 0)
    fire(1, 1)

    def grp2(jj, carry):
        j = jj * 2
        for b in range(2):
            g = j + b
            drain(b)

            def row_body(i, acc_carry, b=b, g=g):
                row = g * _GRP + i
                for k in range(_D // 16):
                    acc = rows_v[b, i, pl.ds(k * 16, 16)]
                    for c in range(1, _CTX):
                        acc = acc + rows_v[b, c * _GRP + i, pl.ds(k * 16, 16)]
                    pooled_v[row, pl.ds(k * 16, 16)] = acc * (1.0 / _CTX)
                for k in range(_D // 16, _DP // 16):
                    pooled_v[row, pl.ds(k * 16, 16)] = zero
                return acc_carry

            lax.fori_loop(0, _GRP, row_body, 0)

            @pl.when(g + 2 < _NGRP)
            def _():
                fire(g + 2, b)
        return carry

    lax.fori_loop(0, _NGRP // 2, grp2, 0)
    pltpu.sync_copy(pooled_v, out_hbm.at[pl.ds(pl.multiple_of(base, 8), _BPW)])


_VB = 1024


def _proj_body(p_ref, w_ref, b_ref, o_ref):
    ot = lax.dot_general(
        w_ref[...], p_ref[...][:, :_D], (((0,), (1,)), ((), ())),
        preferred_element_type=jnp.float32,
    )
    o_ref[...] = ot + b_ref[...].T


@jax.jit
def _project_t(pooled, w_t, bias2d):
    return pl.pallas_call(
        _proj_body,
        grid=(pl.cdiv(_V, _VB),),
        in_specs=[
            pl.BlockSpec((_B, _DP), lambda v: (0, 0)),
            pl.BlockSpec((_D, _VB), lambda v: (0, v)),
            pl.BlockSpec((1, _VB), lambda v: (0, v)),
        ],
        out_specs=pl.BlockSpec((_VB, _B), lambda v: (v, 0)),
        out_shape=jax.ShapeDtypeStruct((_V, _B), jnp.float32),
    )(pooled, w_t, bias2d)


def kernel(context_words_indices, embeddings, linear_w, linear_b):
    idx_flat = jnp.asarray(context_words_indices, jnp.int32).T.reshape(-1)
    pooled128 = _pool(idx_flat, embeddings)
    out_t = _project_t(pooled128, linear_w.T, linear_b.reshape(1, _V))
    return out_t.T
